# baseline (speedup 1.0000x reference)
import functools

import jax
import jax.numpy as jnp
from jax import lax
from jax.experimental import pallas as pl
from jax.experimental.pallas import tpu as pltpu
from jax.experimental.pallas import tpu_sc as plsc

N = 10000
E = 320000
D_IN = 128
D_HID = 512

NC = 2
NS = 16
NT = NC * NS
CH = 125
CHP = 128
ECH = E // CH
CPT = ECH // NT
HALF = CPT // 2
NA = N + 8
STRIPE = (N // NS) // 8 * 8
REM = N - NS * STRIPE

_mesh = plsc.VectorSubcoreMesh(core_axis_name="c", subcore_axis_name="s")


@functools.partial(
    pl.kernel,
    out_type=jax.ShapeDtypeStruct((NC, N, 16), jnp.float32),
    mesh=_mesh,
    scratch_types=[
        pltpu.VMEM((CHP, 16), jnp.float32),
        pltpu.VMEM((CPT, CHP), jnp.int32),
        pltpu.VMEM_SHARED((NA, 16), jnp.float32),
        pltpu.SemaphoreType.DMA,
    ],
)
def _degree_kernel(edge_hbm, zeros16_hbm, deg_hbm, ones_v, dst_v, deg_sh, sem):
    c = lax.axis_index("c")
    s = lax.axis_index("s")
    t = c * NS + s

    def init_ones(r, carry):
        ones_v[r, :] = jnp.ones((16,), jnp.float32)
        return carry

    lax.fori_loop(0, CHP, init_ones, 0)

    pltpu.sync_copy(edge_hbm.at[1, pl.ds(t * CPT, CPT)], dst_v)

    pltpu.sync_copy(zeros16_hbm.at[pl.ds(s * STRIPE, STRIPE)],
                    deg_sh.at[pl.ds(s * STRIPE, STRIPE)])

    @pl.when(s == NS - 1)
    def _():
        pltpu.sync_copy(zeros16_hbm.at[pl.ds(NS * STRIPE, REM)],
                        deg_sh.at[pl.ds(NS * STRIPE, REM)])

    plsc.subcore_barrier()

    _W = 8

    def fire(k, carry):
        pltpu.async_copy(ones_v, deg_sh.at[dst_v.at[k]], sem, add=True)

        @pl.when(k >= _W)
        def _():
            pltpu.make_async_copy(ones_v, deg_sh.at[dst_v.at[k - _W]],
                                  sem).wait()

        return carry

    lax.fori_loop(0, CPT, fire, 0)

    def drain(k, carry):
        pltpu.make_async_copy(ones_v, deg_sh.at[dst_v.at[k]], sem).wait()
        return carry

    lax.fori_loop(CPT - _W, CPT, drain, 0)

    plsc.subcore_barrier()
    pltpu.sync_copy(deg_sh.at[pl.ds(s * STRIPE, STRIPE)],
                    deg_hbm.at[c, pl.ds(s * STRIPE, STRIPE)])

    @pl.when(s == NS - 1)
    def _():
        pltpu.sync_copy(deg_sh.at[pl.ds(NS * STRIPE, REM)],
                        deg_hbm.at[c, pl.ds(NS * STRIPE, REM)])


@functools.partial(
    pl.kernel,
    out_type=jax.ShapeDtypeStruct((NC, N, D_IN), jnp.float32),
    mesh=_mesh,
    scratch_types=[
        pltpu.VMEM((HALF, CHP), jnp.int32),
        pltpu.VMEM((HALF, CHP), jnp.int32),
        [pltpu.VMEM((CHP, D_IN), jnp.float32) for _ in range(2)],
        pltpu.VMEM_SHARED((NA, D_IN), jnp.float32),
        [pltpu.SemaphoreType.DMA for _ in range(2)],
        [pltpu.SemaphoreType.DMA for _ in range(2)],
    ],
)
def _scatter_kernel(edge_hbm, y_hbm, zeros_hbm, out_hbm,
                    src_v, dst_v, rows, acc_sh, gsem, ssem):
    c = lax.axis_index("c")
    s = lax.axis_index("s")
    t = c * NS + s

    pltpu.sync_copy(zeros_hbm.at[pl.ds(s * STRIPE, STRIPE)],
                    acc_sh.at[pl.ds(s * STRIPE, STRIPE)])

    @pl.when(s == NS - 1)
    def _():
        pltpu.sync_copy(zeros_hbm.at[pl.ds(NS * STRIPE, REM)],
                        acc_sh.at[pl.ds(NS * STRIPE, REM)])

    plsc.subcore_barrier()

    for st in range(2):
        base = t * CPT + st * HALF
        pltpu.sync_copy(edge_hbm.at[0, pl.ds(base, HALF)], src_v)
        pltpu.sync_copy(edge_hbm.at[1, pl.ds(base, HALF)], dst_v)

        pltpu.async_copy(y_hbm.at[src_v.at[0]], rows[0], gsem[0])

        def step(j, carry):
            for r in range(2):
                j2 = 2 * j + r
                pltpu.make_async_copy(y_hbm.at[src_v.at[j2]], rows[r],
                                      gsem[r]).wait()
                pltpu.async_copy(rows[r], acc_sh.at[dst_v.at[j2]], ssem[r],
                                 add=True)

                @pl.when(j2 >= 1)
                def _():
                    pltpu.make_async_copy(rows[1 - r],
                                          acc_sh.at[dst_v.at[j2 - 1]],
                                          ssem[1 - r]).wait()

                @pl.when(j2 < HALF - 1)
                def _():
                    pltpu.async_copy(y_hbm.at[src_v.at[j2 + 1]], rows[1 - r],
                                     gsem[1 - r])

            return carry

        lax.fori_loop(0, HALF // 2, step, 0)
        pltpu.make_async_copy(rows[1], acc_sh.at[dst_v.at[HALF - 1]],
                              ssem[1]).wait()

    plsc.subcore_barrier()
    pltpu.sync_copy(acc_sh.at[pl.ds(s * STRIPE, STRIPE)],
                    out_hbm.at[c, pl.ds(s * STRIPE, STRIPE)])

    @pl.when(s == NS - 1)
    def _():
        pltpu.sync_copy(acc_sh.at[pl.ds(NS * STRIPE, REM)],
                        out_hbm.at[c, pl.ds(NS * STRIPE, REM)])


_RB = 1000


def _scale_body(deg_ref, x_ref, y_ref):
    d16 = deg_ref[0] + deg_ref[1]
    deg = jnp.sum(d16, axis=1) * (1.0 / 16.0) + 1.0
    dinv = lax.rsqrt(deg)
    y_ref[...] = x_ref[...] * dinv[:, None]


def _scale(deg16, x):
    return pl.pallas_call(
        _scale_body,
        grid=(N // _RB,),
        in_specs=[
            pl.BlockSpec((NC, _RB, 16), lambda i: (0, i, 0)),
            pl.BlockSpec((_RB, D_IN), lambda i: (i, 0)),
        ],
        out_specs=pl.BlockSpec((_RB, D_IN), lambda i: (i, 0)),
        out_shape=jax.ShapeDtypeStruct((N, D_IN), jnp.float32),
    )(deg16, x)


def _mlp_body(p_ref, y_ref, deg_ref, w1_ref, b1_ref, w2_ref, b2_ref,
              w3_ref, b3_ref, out_ref):
    d16 = deg_ref[0] + deg_ref[1]
    deg = jnp.sum(d16, axis=1) * (1.0 / 16.0) + 1.0
    dinv = lax.rsqrt(deg)
    agg = (p_ref[0] + p_ref[1] + y_ref[...]) * dinv[:, None]
    h = jnp.dot(agg, w1_ref[...], preferred_element_type=jnp.float32,
                precision=lax.Precision.HIGHEST) + b1_ref[...]
    h = jnp.where(h > 0, h, 0.2 * h)
    h = jnp.dot(h, w2_ref[...], preferred_element_type=jnp.float32,
                precision=lax.Precision.HIGHEST) + b2_ref[...]
    h = jnp.where(h > 0, h, 0.2 * h)
    out_ref[...] = jnp.dot(h, w3_ref[...], preferred_element_type=jnp.float32,
                           precision=lax.Precision.HIGHEST) + b3_ref[...]


def _mlp(parts, y, deg16, W1, b1, W2, b2, W3, b3):
    return pl.pallas_call(
        _mlp_body,
        grid=(N // _RB,),
        in_specs=[
            pl.BlockSpec((NC, _RB, D_IN), lambda i: (0, i, 0)),
            pl.BlockSpec((_RB, D_IN), lambda i: (i, 0)),
            pl.BlockSpec((NC, _RB, 16), lambda i: (0, i, 0)),
            pl.BlockSpec((D_IN, D_HID), lambda i: (0, 0)),
            pl.BlockSpec((D_HID,), lambda i: (0,)),
            pl.BlockSpec((D_HID, D_HID), lambda i: (0, 0)),
            pl.BlockSpec((D_HID,), lambda i: (0,)),
            pl.BlockSpec((D_HID, 1), lambda i: (0, 0)),
            pl.BlockSpec((1,), lambda i: (0,)),
        ],
        out_specs=pl.BlockSpec((_RB, 1), lambda i: (i, 0)),
        out_shape=jax.ShapeDtypeStruct((N, 1), jnp.float32),
    )(parts, y, deg16, W1, b1, W2, b2, W3, b3)


def kernel(input_embd, edge_index, W1, b1, W2, b2, W3, b3):
    e3 = edge_index.reshape(2, ECH, CH)
    pad = jnp.stack([jnp.zeros((ECH, CHP - CH), jnp.int32),
                     jnp.full((ECH, CHP - CH), N, jnp.int32)])
    edge3 = jnp.concatenate([e3, pad], axis=2)
    zeros16 = jnp.zeros((N, 16), jnp.float32)
    zeros128 = jnp.zeros((N, D_IN), jnp.float32)
    deg16 = _degree_kernel(edge3, zeros16)
    y = _scale(deg16, input_embd)
    parts = _scatter_kernel(edge3, y, zeros128)
    return _mlp(parts, y, deg16, W1, b1, W2, b2, W3, b3)
